# baseline (speedup 1.0000x reference)
import functools

import jax
import jax.numpy as jnp
from jax import lax
from jax.experimental import pallas as pl
from jax.experimental.pallas import tpu as pltpu
from jax.experimental.pallas import tpu_sc as plsc

VOCAB = 100000
EMBED = 128
BATCH = 4096
SEQ = 50

NC = 2
NS = 16
NW = NC * NS
BPW = BATCH // NW
L = 16
NCH = EMBED // L
NBUF = 6

_mesh = plsc.VectorSubcoreMesh(core_axis_name="c", subcore_axis_name="s")


def _sc_body(x1_hbm, x2_hbm, emb_hbm, part_hbm, *refs):
  idx1_v = refs[0]
  idx2_v = refs[1]
  rows = list(refs[2:2 + NBUF])
  part_v = refs[2 + NBUF]
  sems = list(refs[3 + NBUF:3 + 2 * NBUF])

  wid = lax.axis_index("s") * NC + lax.axis_index("c")
  base = wid * BPW

  pltpu.sync_copy(x1_hbm.at[pl.ds(base, BPW)], idx1_v)
  pltpu.sync_copy(x2_hbm.at[pl.ds(base, BPW)], idx2_v)

  def gathers(b, j):
    return (
        pltpu.make_async_copy(
            emb_hbm.at[idx1_v.at[b]], rows[j].at[pl.ds(0, SEQ)], sems[j]),
        pltpu.make_async_copy(
            emb_hbm.at[idx2_v.at[b]], rows[j].at[pl.ds(SEQ, SEQ)], sems[j]),
    )

  def start(b, j):
    c1, c2 = gathers(b, j)
    c1.start()
    c2.start()

  def wait(b, j):
    c1, c2 = gathers(b, j)
    c1.wait()
    c2.wait()

  def accumulate(rows_v, b):
    def rbody(r, accs):
      new = []
      for c in range(NCH):
        new.append(accs[c] + rows_v[r, pl.ds(c * L, L)])
      for c in range(NCH):
        new.append(accs[NCH + c] + rows_v[SEQ + r, pl.ds(c * L, L)])
      return tuple(new)

    init = tuple(
        [rows_v[0, pl.ds(c * L, L)] for c in range(NCH)]
        + [rows_v[SEQ, pl.ds(c * L, L)] for c in range(NCH)])
    accs = lax.fori_loop(1, SEQ, rbody, init, unroll=2)
    dotv = accs[0] * accs[NCH]
    ss1v = accs[0] * accs[0]
    ss2v = accs[NCH] * accs[NCH]
    for c in range(1, NCH):
      dotv += accs[c] * accs[NCH + c]
      ss1v += accs[c] * accs[c]
      ss2v += accs[NCH + c] * accs[NCH + c]
    part_v[b, pl.ds(0, L)] = dotv
    part_v[b, pl.ds(L, L)] = ss1v
    part_v[b, pl.ds(2 * L, L)] = ss2v

  for j in range(NBUF - 1):
    start(j, j)

  def group(i, carry):
    b0 = NBUF * i
    for j in range(NBUF):
      b = b0 + j
      wait(b, j)

      @pl.when(b + NBUF - 1 < BPW)
      def _():
        start(b + NBUF - 1, (j + NBUF - 1) % NBUF)

      accumulate(rows[j], b)

    return carry

  n_groups = BPW // NBUF
  lax.fori_loop(0, n_groups, group, 0)
  for b in range(n_groups * NBUF, BPW):
    wait(b, b % NBUF)
    accumulate(rows[b % NBUF], b)

  pltpu.sync_copy(part_v, part_hbm.at[pl.ds(base, BPW)])


_sc_pool = functools.partial(
    pl.kernel,
    out_type=jax.ShapeDtypeStruct((BATCH, 3 * L), jnp.float32),
    mesh=_mesh,
    scratch_types=(
        [pltpu.VMEM((BPW, SEQ), jnp.int32),
         pltpu.VMEM((BPW, SEQ), jnp.int32)]
        + [pltpu.VMEM((2 * SEQ, EMBED), jnp.float32) for _ in range(NBUF)]
        + [pltpu.VMEM((BPW, 3 * L), jnp.float32)]
        + [pltpu.SemaphoreType.DMA for _ in range(NBUF)]
    ),
)(_sc_body)


def _tc_body(part_ref, w_ref, b_ref, out_ref):
  part = part_ref[...]
  dot = jnp.sum(part[:, 0:L], axis=1, keepdims=True) * (1.0 / (SEQ * SEQ))
  ss1 = jnp.sum(part[:, L:2 * L], axis=1, keepdims=True)
  ss2 = jnp.sum(part[:, 2 * L:3 * L], axis=1, keepdims=True)
  eps = 1e-8
  n1 = jnp.maximum(jnp.sqrt(ss1) * (1.0 / SEQ), eps)
  n2 = jnp.maximum(jnp.sqrt(ss2) * (1.0 / SEQ), eps)
  cos = dot / (n1 * n2)
  out_ref[...] = cos * w_ref[...] + b_ref[...]


def _tc_epilogue(part, w_t, b_t):
  return pl.pallas_call(
      _tc_body,
      out_shape=jax.ShapeDtypeStruct((BATCH, 2), jnp.float32),
  )(part, w_t, b_t)


def kernel(x1, x2, embedding, fc_w, fc_b):
  part = _sc_pool(x1.astype(jnp.int32), x2.astype(jnp.int32), embedding)
  w_t = fc_w.reshape(1, 2)
  b_t = fc_b.reshape(1, 2)
  return _tc_epilogue(part, w_t, b_t)


# speedup vs baseline: 1.0182x; 1.0182x over previous
import functools

import jax
import jax.numpy as jnp
from jax import lax
from jax.experimental import pallas as pl
from jax.experimental.pallas import tpu as pltpu
from jax.experimental.pallas import tpu_sc as plsc

VOCAB = 100000
EMBED = 128
BATCH = 4096
SEQ = 50

NC = 2
NS = 16
NW = NC * NS
BPW = BATCH // NW
L = 16
NCH = EMBED // L
NBUF = 6

_mesh = plsc.VectorSubcoreMesh(core_axis_name="c", subcore_axis_name="s")


def _sc_body(xcat_hbm, emb_hbm, part_hbm, *refs):
  idx_v = refs[0]
  rows = list(refs[1:1 + NBUF])
  part_v = refs[1 + NBUF]
  sems = list(refs[2 + NBUF:2 + 2 * NBUF])

  wid = lax.axis_index("s") * NC + lax.axis_index("c")
  base = wid * BPW

  pltpu.sync_copy(xcat_hbm.at[pl.ds(base, BPW)], idx_v)

  def gather(b, j):
    return pltpu.make_async_copy(emb_hbm.at[idx_v.at[b]], rows[j], sems[j])

  def start(b, j):
    gather(b, j).start()

  def wait(b, j):
    gather(b, j).wait()

  def accumulate(rows_v, b):
    def rbody(r, accs):
      new = []
      for c in range(NCH):
        new.append(accs[c] + rows_v[r, pl.ds(c * L, L)])
      for c in range(NCH):
        new.append(accs[NCH + c] + rows_v[SEQ + r, pl.ds(c * L, L)])
      return tuple(new)

    init = tuple(
        [rows_v[0, pl.ds(c * L, L)] for c in range(NCH)]
        + [rows_v[SEQ, pl.ds(c * L, L)] for c in range(NCH)])
    accs = lax.fori_loop(1, SEQ, rbody, init, unroll=2)
    dotv = accs[0] * accs[NCH]
    ss1v = accs[0] * accs[0]
    ss2v = accs[NCH] * accs[NCH]
    for c in range(1, NCH):
      dotv += accs[c] * accs[NCH + c]
      ss1v += accs[c] * accs[c]
      ss2v += accs[NCH + c] * accs[NCH + c]
    part_v[b, pl.ds(0, L)] = dotv
    part_v[b, pl.ds(L, L)] = ss1v
    part_v[b, pl.ds(2 * L, L)] = ss2v

  for j in range(NBUF - 1):
    start(j, j)

  def group(i, carry):
    b0 = NBUF * i
    for j in range(NBUF):
      b = b0 + j
      wait(b, j)

      @pl.when(b + NBUF - 1 < BPW)
      def _():
        start(b + NBUF - 1, (j + NBUF - 1) % NBUF)

      accumulate(rows[j], b)

    return carry

  n_groups = BPW // NBUF
  lax.fori_loop(0, n_groups, group, 0)
  for b in range(n_groups * NBUF, BPW):
    wait(b, b % NBUF)
    accumulate(rows[b % NBUF], b)

  pltpu.sync_copy(part_v, part_hbm.at[pl.ds(base, BPW)])


_sc_pool = functools.partial(
    pl.kernel,
    out_type=jax.ShapeDtypeStruct((BATCH, 3 * L), jnp.float32),
    mesh=_mesh,
    scratch_types=(
        [pltpu.VMEM((BPW, 2 * SEQ), jnp.int32)]
        + [pltpu.VMEM((2 * SEQ, EMBED), jnp.float32) for _ in range(NBUF)]
        + [pltpu.VMEM((BPW, 3 * L), jnp.float32)]
        + [pltpu.SemaphoreType.DMA for _ in range(NBUF)]
    ),
)(_sc_body)


def _tc_body(part_ref, w_ref, b_ref, out_ref):
  part = part_ref[...]
  dot = jnp.sum(part[:, 0:L], axis=1, keepdims=True) * (1.0 / (SEQ * SEQ))
  ss1 = jnp.sum(part[:, L:2 * L], axis=1, keepdims=True)
  ss2 = jnp.sum(part[:, 2 * L:3 * L], axis=1, keepdims=True)
  eps = 1e-8
  n1 = jnp.maximum(jnp.sqrt(ss1) * (1.0 / SEQ), eps)
  n2 = jnp.maximum(jnp.sqrt(ss2) * (1.0 / SEQ), eps)
  cos = dot / (n1 * n2)
  out_ref[...] = cos * w_ref[...] + b_ref[...]


def _tc_epilogue(part, w_t, b_t):
  return pl.pallas_call(
      _tc_body,
      out_shape=jax.ShapeDtypeStruct((BATCH, 2), jnp.float32),
  )(part, w_t, b_t)


def kernel(x1, x2, embedding, fc_w, fc_b):
  xcat = jnp.concatenate(
      [x1.astype(jnp.int32), x2.astype(jnp.int32)], axis=1)
  part = _sc_pool(xcat, embedding)
  w_t = fc_w.reshape(1, 2)
  b_t = fc_b.reshape(1, 2)
  return _tc_epilogue(part, w_t, b_t)
